# Initial kernel scaffold; baseline (speedup 1.0000x reference)
import jax
import jax.numpy as jnp
from jax.experimental import pallas as pl


def kernel(x, mask, table):
    raise NotImplementedError("write your pallas kernel here")


# speedup vs baseline: 2.2728x; 2.2728x over previous
import jax
import jax.numpy as jnp
from jax.experimental import pallas as pl


_S_BLK = 512


def _body(mask_ref, table_ref, out_ref):
    t = table_ref[...]
    m = mask_ref[...]
    out_ref[...] = m[:, :, None] * t[None, :, :]


def kernel(x, mask, table):
    batch, seq_len, dim = x.shape
    grid = (seq_len // _S_BLK,)
    return pl.pallas_call(
        _body,
        grid=grid,
        in_specs=[
            pl.BlockSpec((batch, _S_BLK), lambda i: (0, i)),
            pl.BlockSpec((_S_BLK, dim), lambda i: (i, 0)),
        ],
        out_specs=pl.BlockSpec((batch, _S_BLK, dim), lambda i: (0, i, 0)),
        out_shape=jax.ShapeDtypeStruct((batch, seq_len, dim), jnp.float32),
    )(mask, table[:seq_len])
